# Initial kernel scaffold; baseline (speedup 1.0000x reference)
import jax
import jax.numpy as jnp
from jax.experimental import pallas as pl


def kernel(node_embeddings, edge_index, relation_type, rel_weight):
    raise NotImplementedError("write your pallas kernel here")


# speedup vs baseline: 2.1204x; 2.1204x over previous
import functools

import jax
import jax.numpy as jnp
from jax import lax
from jax.experimental import pallas as pl
from jax.experimental.pallas import tpu as pltpu
from jax.experimental.pallas import tpu_sc as plsc

_N_NODES = 10000
_D = 128
_N_REL = 16
_E = 320000

_L = 16
_NC = 2
_NS = 16
_NW = _NC * _NS
_EW = _E // _NW
_C = 80
_NCHUNK = _EW // _C
_NGROUP = _C // _L
_DCH = _D // _L


def _sc_body(node_hbm, head_hbm, tail_hbm, rel_hbm, relw_hbm, out_hbm,
             hidx_v, tidx_v, ridx_v, hrows_v, trows_v, rrows_v, out_v,
             sem_h, sem_t, sem_r):
    wid = lax.axis_index("s") * _NC + lax.axis_index("c")
    base = wid * _EW
    lane = lax.iota(jnp.int32, _L)
    perms = [jnp.bitwise_xor(lane, jnp.int32(1 << b)) for b in range(4)]
    gdn = lax.GatherDimensionNumbers(
        offset_dims=(), collapsed_slice_dims=(0,), start_index_map=(0,))

    def _permute(x, p):
        return lax.gather(x, p[:, None], gdn, (1,),
                          mode=lax.GatherScatterMode.PROMISE_IN_BOUNDS)

    def chunk_body(i, carry):
        off = base + i * _C
        pltpu.sync_copy(head_hbm.at[pl.ds(off, _C)], hidx_v)
        pltpu.sync_copy(tail_hbm.at[pl.ds(off, _C)], tidx_v)
        pltpu.sync_copy(rel_hbm.at[pl.ds(off, _C)], ridx_v)
        cph = pltpu.async_copy(node_hbm.at[hidx_v], hrows_v, sem_h)
        cpt = pltpu.async_copy(node_hbm.at[tidx_v], trows_v, sem_t)
        cpr = pltpu.async_copy(relw_hbm.at[ridx_v], rrows_v, sem_r)
        cph.wait()
        cpt.wait()
        cpr.wait()

        def group_body(g, gcarry):
            scores = jnp.zeros((_L,), jnp.float32)
            for k in range(_L):
                row = g * _L + k
                acc = (hrows_v[row, pl.ds(0, _L)]
                       * rrows_v[row, pl.ds(0, _L)]
                       * trows_v[row, pl.ds(0, _L)])
                for dd in range(1, _DCH):
                    acc = acc + (hrows_v[row, pl.ds(dd * _L, _L)]
                                 * rrows_v[row, pl.ds(dd * _L, _L)]
                                 * trows_v[row, pl.ds(dd * _L, _L)])
                for p in perms:
                    acc = acc + _permute(acc, p)
                scores = jnp.where(lane == k, acc, scores)
            out_v[pl.ds(g * _L, _L)] = scores * (1.0 / _D)
            return gcarry

        lax.fori_loop(0, _NGROUP, group_body, 0)
        pltpu.sync_copy(out_v, out_hbm.at[pl.ds(off, _C)])
        return carry

    lax.fori_loop(0, _NCHUNK, chunk_body, 0)


@functools.partial(jax.jit, static_argnums=())
def _sc_score(node_embeddings, head, tail, rel_type, rel_weight):
    mesh = plsc.VectorSubcoreMesh(core_axis_name="c", subcore_axis_name="s")
    kfn = functools.partial(
        pl.kernel,
        mesh=mesh,
        out_type=jax.ShapeDtypeStruct((_E,), jnp.float32),
        scratch_types=[
            pltpu.VMEM((_C,), jnp.int32),
            pltpu.VMEM((_C,), jnp.int32),
            pltpu.VMEM((_C,), jnp.int32),
            pltpu.VMEM((_C, _D), jnp.float32),
            pltpu.VMEM((_C, _D), jnp.float32),
            pltpu.VMEM((_C, _D), jnp.float32),
            pltpu.VMEM((_C,), jnp.float32),
            pltpu.SemaphoreType.DMA,
            pltpu.SemaphoreType.DMA,
            pltpu.SemaphoreType.DMA,
        ],
    )(_sc_body)
    return kfn(node_embeddings, head, tail, rel_type, rel_weight)


def kernel(node_embeddings, edge_index, relation_type, rel_weight):
    head = edge_index[0]
    tail = edge_index[1]
    rel_type = relation_type.astype(jnp.int32)
    return _sc_score(node_embeddings, head, tail, rel_type, rel_weight)


# speedup vs baseline: 2.3800x; 1.1224x over previous
import functools

import jax
import jax.numpy as jnp
from jax import lax
from jax.experimental import pallas as pl
from jax.experimental.pallas import tpu as pltpu
from jax.experimental.pallas import tpu_sc as plsc

_N_NODES = 10000
_D = 128
_N_REL = 16
_E = 320000

_L = 16
_NC = 2
_NS = 16
_NW = _NC * _NS
_EW = _E // _NW
_C = 80
_NCHUNK = _EW // _C
_NGROUP = _C // _L
_DCH = _D // _L


def _sc_body(node_hbm, head_hbm, tail_hbm, rel_hbm, relw_hbm, out_hbm,
             hidx_v, tidx_v, ridx_v,
             hrows0, trows0, rrows0, hrows1, trows1, rrows1,
             out_v,
             sh0, st0, sr0, sh1, st1, sr1):
    wid = lax.axis_index("s") * _NC + lax.axis_index("c")
    lane = lax.iota(jnp.int32, _L)
    perms = [jnp.bitwise_xor(lane, jnp.int32(1 << b)) for b in range(4)]
    gdn = lax.GatherDimensionNumbers(
        offset_dims=(), collapsed_slice_dims=(0,), start_index_map=(0,))

    def _permute(x, p):
        return lax.gather(x, p[:, None], gdn, (1,),
                          mode=lax.GatherScatterMode.PROMISE_IN_BOUNDS)

    bufs = ((hrows0, trows0, rrows0, sh0, st0, sr0),
            (hrows1, trows1, rrows1, sh1, st1, sr1))

    pltpu.sync_copy(head_hbm.at[wid], hidx_v)
    pltpu.sync_copy(tail_hbm.at[wid], tidx_v)
    pltpu.sync_copy(rel_hbm.at[wid], ridx_v)

    def fire(i, b):
        hr, tr, rr, sh, st, sr = bufs[b]
        pltpu.async_copy(node_hbm.at[hidx_v.at[i]], hr, sh)
        pltpu.async_copy(node_hbm.at[tidx_v.at[i]], tr, st)
        pltpu.async_copy(relw_hbm.at[ridx_v.at[i]], rr, sr)

    def wait(i, b):
        hr, tr, rr, sh, st, sr = bufs[b]
        pltpu.make_async_copy(node_hbm.at[hidx_v.at[i]], hr, sh).wait()
        pltpu.make_async_copy(node_hbm.at[tidx_v.at[i]], tr, st).wait()
        pltpu.make_async_copy(relw_hbm.at[ridx_v.at[i]], rr, sr).wait()

    def compute(i, b):
        hr, tr, rr = bufs[b][0], bufs[b][1], bufs[b][2]

        def group_body(g, _):
            def edge_body(k, scores):
                row = g * _L + k
                acc = (hr[row, pl.ds(0, _L)]
                       * rr[row, pl.ds(0, _L)]
                       * tr[row, pl.ds(0, _L)])
                for dd in range(1, _DCH):
                    acc = acc + (hr[row, pl.ds(dd * _L, _L)]
                                 * rr[row, pl.ds(dd * _L, _L)]
                                 * tr[row, pl.ds(dd * _L, _L)])
                for p in perms:
                    acc = acc + _permute(acc, p)
                return jnp.where(lane == k, acc, scores)

            scores = lax.fori_loop(0, _L, edge_body, jnp.zeros((_L,), jnp.float32))
            out_v[pl.ds(i * _C + g * _L, _L)] = scores * (1.0 / _D)
            return 0

        lax.fori_loop(0, _NGROUP, group_body, 0)

    fire(0, 0)
    fire(1, 1)
    last = jnp.int32(_NCHUNK - 1)

    def pair_body(u, _):
        for b in range(2):
            i = u * 2 + b
            wait(i, b)
            compute(i, b)
            fire(jnp.minimum(i + 2, last), b)
        return 0

    lax.fori_loop(0, (_NCHUNK - 1) // 2, pair_body, 0)
    wait(last, 0)
    compute(last, 0)
    wait(last, 1)
    pltpu.sync_copy(out_v, out_hbm.at[wid])


@jax.jit
def _sc_score(node_embeddings, head, tail, rel_type, rel_weight):
    mesh = plsc.VectorSubcoreMesh(core_axis_name="c", subcore_axis_name="s")
    kfn = functools.partial(
        pl.kernel,
        mesh=mesh,
        out_type=jax.ShapeDtypeStruct((_NW, _EW), jnp.float32),
        scratch_types=[
            pltpu.VMEM((_NCHUNK, _C), jnp.int32),
            pltpu.VMEM((_NCHUNK, _C), jnp.int32),
            pltpu.VMEM((_NCHUNK, _C), jnp.int32),
            pltpu.VMEM((_C, _D), jnp.float32),
            pltpu.VMEM((_C, _D), jnp.float32),
            pltpu.VMEM((_C, _D), jnp.float32),
            pltpu.VMEM((_C, _D), jnp.float32),
            pltpu.VMEM((_C, _D), jnp.float32),
            pltpu.VMEM((_C, _D), jnp.float32),
            pltpu.VMEM((_EW,), jnp.float32),
            pltpu.SemaphoreType.DMA,
            pltpu.SemaphoreType.DMA,
            pltpu.SemaphoreType.DMA,
            pltpu.SemaphoreType.DMA,
            pltpu.SemaphoreType.DMA,
            pltpu.SemaphoreType.DMA,
        ],
    )(_sc_body)
    return kfn(node_embeddings, head, tail, rel_type, rel_weight)


def kernel(node_embeddings, edge_index, relation_type, rel_weight):
    head = edge_index[0].reshape(_NW, _NCHUNK, _C)
    tail = edge_index[1].reshape(_NW, _NCHUNK, _C)
    rel_type = relation_type.astype(jnp.int32).reshape(_NW, _NCHUNK, _C)
    out = _sc_score(node_embeddings, head, tail, rel_type, rel_weight)
    return out.reshape(_E)


# speedup vs baseline: 2.3826x; 1.0011x over previous
import functools

import jax
import jax.numpy as jnp
from jax import lax
from jax.experimental import pallas as pl
from jax.experimental.pallas import tpu as pltpu
from jax.experimental.pallas import tpu_sc as plsc

_N_NODES = 10000
_D = 128
_N_REL = 16
_E = 320000

_L = 16
_NC = 2
_NS = 16
_NW = _NC * _NS
_EW = _E // _NW
_C = 80
_NCHUNK = _EW // _C
_NGROUP = _C // _L
_DCH = _D // _L


def _sc_body(node_hbm, head_hbm, tail_hbm, rel_hbm, relw_hbm, out_hbm,
             hidx_v, tidx_v, ridx_v,
             hrows0, trows0, rrows0, hrows1, trows1, rrows1,
             out_v,
             sh0, st0, sr0, sh1, st1, sr1):
    wid = lax.axis_index("s") * _NC + lax.axis_index("c")
    lane = lax.iota(jnp.int32, _L)
    perms = [jnp.bitwise_xor(lane, jnp.int32(1 << b)) for b in range(4)]
    gdn = lax.GatherDimensionNumbers(
        offset_dims=(), collapsed_slice_dims=(0,), start_index_map=(0,))

    def _permute(x, p):
        return lax.gather(x, p[:, None], gdn, (1,),
                          mode=lax.GatherScatterMode.PROMISE_IN_BOUNDS)

    bufs = ((hrows0, trows0, rrows0, sh0, st0, sr0),
            (hrows1, trows1, rrows1, sh1, st1, sr1))

    pltpu.sync_copy(head_hbm.at[wid], hidx_v)
    pltpu.sync_copy(tail_hbm.at[wid], tidx_v)
    pltpu.sync_copy(rel_hbm.at[wid], ridx_v)

    def fire(i, b):
        hr, tr, rr, sh, st, sr = bufs[b]
        pltpu.async_copy(node_hbm.at[hidx_v.at[i]], hr, sh)
        pltpu.async_copy(node_hbm.at[tidx_v.at[i]], tr, st)
        pltpu.async_copy(relw_hbm.at[ridx_v.at[i]], rr, sr)

    def wait(i, b):
        hr, tr, rr, sh, st, sr = bufs[b]
        pltpu.make_async_copy(node_hbm.at[hidx_v.at[i]], hr, sh).wait()
        pltpu.make_async_copy(node_hbm.at[tidx_v.at[i]], tr, st).wait()
        pltpu.make_async_copy(relw_hbm.at[ridx_v.at[i]], rr, sr).wait()

    def compute(i, b):
        hr, tr, rr = bufs[b][0], bufs[b][1], bufs[b][2]

        def group_body(g, _):
            def quad_body(q, scores):
                for j in range(4):
                    k = q * 4 + j
                    row = g * _L + k
                    ps = [hr[row, pl.ds(dd * _L, _L)]
                          * rr[row, pl.ds(dd * _L, _L)]
                          * tr[row, pl.ds(dd * _L, _L)]
                          for dd in range(_DCH)]
                    while len(ps) > 1:
                        ps = [ps[m] + ps[m + 1] for m in range(0, len(ps), 2)]
                    acc = ps[0]
                    for p in perms:
                        acc = acc + _permute(acc, p)
                    scores = jnp.where(lane == k, acc, scores)
                return scores

            scores = lax.fori_loop(0, 4, quad_body, jnp.zeros((_L,), jnp.float32))
            out_v[pl.ds(i * _C + g * _L, _L)] = scores * (1.0 / _D)
            return 0

        lax.fori_loop(0, _NGROUP, group_body, 0)

    fire(0, 0)
    fire(1, 1)
    last = jnp.int32(_NCHUNK - 1)

    def pair_body(u, _):
        for b in range(2):
            i = u * 2 + b
            wait(i, b)
            compute(i, b)
            fire(jnp.minimum(i + 2, last), b)
        return 0

    lax.fori_loop(0, (_NCHUNK - 1) // 2, pair_body, 0)
    wait(last, 0)
    compute(last, 0)
    wait(last, 1)
    pltpu.sync_copy(out_v, out_hbm.at[wid])


@jax.jit
def _sc_score(node_embeddings, head, tail, rel_type, rel_weight):
    mesh = plsc.VectorSubcoreMesh(core_axis_name="c", subcore_axis_name="s")
    kfn = functools.partial(
        pl.kernel,
        mesh=mesh,
        out_type=jax.ShapeDtypeStruct((_NW, _EW), jnp.float32),
        scratch_types=[
            pltpu.VMEM((_NCHUNK, _C), jnp.int32),
            pltpu.VMEM((_NCHUNK, _C), jnp.int32),
            pltpu.VMEM((_NCHUNK, _C), jnp.int32),
            pltpu.VMEM((_C, _D), jnp.float32),
            pltpu.VMEM((_C, _D), jnp.float32),
            pltpu.VMEM((_C, _D), jnp.float32),
            pltpu.VMEM((_C, _D), jnp.float32),
            pltpu.VMEM((_C, _D), jnp.float32),
            pltpu.VMEM((_C, _D), jnp.float32),
            pltpu.VMEM((_EW,), jnp.float32),
            pltpu.SemaphoreType.DMA,
            pltpu.SemaphoreType.DMA,
            pltpu.SemaphoreType.DMA,
            pltpu.SemaphoreType.DMA,
            pltpu.SemaphoreType.DMA,
            pltpu.SemaphoreType.DMA,
        ],
    )(_sc_body)
    return kfn(node_embeddings, head, tail, rel_type, rel_weight)


def kernel(node_embeddings, edge_index, relation_type, rel_weight):
    head = edge_index[0].reshape(_NW, _NCHUNK, _C)
    tail = edge_index[1].reshape(_NW, _NCHUNK, _C)
    rel_type = relation_type.astype(jnp.int32).reshape(_NW, _NCHUNK, _C)
    out = _sc_score(node_embeddings, head, tail, rel_type, rel_weight)
    return out.reshape(_E)


# speedup vs baseline: 8.8024x; 3.6945x over previous
import functools

import jax
import jax.numpy as jnp
from jax import lax
from jax.experimental import pallas as pl
from jax.experimental.pallas import tpu as pltpu
from jax.experimental.pallas import tpu_sc as plsc

_N_NODES = 10000
_D = 128
_N_REL = 16
_E = 320000

_L = 16
_NC = 2
_NS = 16
_NW = _NC * _NS
_EW = _E // _NW
_C = 40
_NCHUNK = _EW // _C
_DCH = _D // _L
_OPAD = _EW + _L


def _sc_body(node_hbm, idx_hbm, relw_hbm, out_hbm,
             node_sh, relw_sh,
             ibuf0, ibuf1,
             hrows0, trows0, rrows0, hrows1, trows1, rrows1,
             out_v,
             si0, si1, sh0, st0, sr0, sh1, st1, sr1):
    wid = lax.axis_index("s") * _NC + lax.axis_index("c")
    sid = lax.axis_index("s")
    lane = lax.iota(jnp.int32, _L)
    perms = [jnp.bitwise_xor(lane, jnp.int32(1 << b)) for b in range(4)]
    gdn = lax.GatherDimensionNumbers(
        offset_dims=(), collapsed_slice_dims=(0,), start_index_map=(0,))

    def _permute(x, p):
        return lax.gather(x, p[:, None], gdn, (1,),
                          mode=lax.GatherScatterMode.PROMISE_IN_BOUNDS)

    ibufs = (ibuf0, ibuf1)
    isems = (si0, si1)
    rows = ((hrows0, trows0, rrows0), (hrows1, trows1, rrows1))
    gsems = ((sh0, st0, sr0), (sh1, st1, sr1))

    @pl.when(sid == 0)
    def _():
        pltpu.sync_copy(node_hbm, node_sh)
        pltpu.sync_copy(relw_hbm, relw_sh)

    plsc.subcore_barrier()

    def fire_idx(i, b):
        pltpu.async_copy(idx_hbm.at[wid, i], ibufs[b], isems[b])

    def wait_idx(i, b):
        pltpu.make_async_copy(idx_hbm.at[wid, i], ibufs[b], isems[b]).wait()

    def fire_g(b):
        hr, tr, rr = rows[b]
        sh, st, sr = gsems[b]
        ib = ibufs[b]
        pltpu.async_copy(node_sh.at[ib.at[0]], hr, sh)
        pltpu.async_copy(node_sh.at[ib.at[1]], tr, st)
        pltpu.async_copy(relw_sh.at[ib.at[2]], rr, sr)

    def wait_g(b):
        hr, tr, rr = rows[b]
        sh, st, sr = gsems[b]
        ib = ibufs[b]
        pltpu.make_async_copy(node_sh.at[ib.at[0]], hr, sh).wait()
        pltpu.make_async_copy(node_sh.at[ib.at[1]], tr, st).wait()
        pltpu.make_async_copy(relw_sh.at[ib.at[2]], rr, sr).wait()

    def compute(i, b):
        hr, tr, rr = rows[b]

        def edge4(base_row, q, scores):
            for j in range(4):
                k = q * 4 + j
                row = base_row + k
                ps = [hr[row, pl.ds(dd * _L, _L)]
                      * rr[row, pl.ds(dd * _L, _L)]
                      * tr[row, pl.ds(dd * _L, _L)]
                      for dd in range(_DCH)]
                while len(ps) > 1:
                    ps = [ps[m] + ps[m + 1] for m in range(0, len(ps), 2)]
                acc = ps[0]
                for p in perms:
                    acc = acc + _permute(acc, p)
                scores = jnp.where(lane == k, acc, scores)
            return scores

        def group_body(g, _):
            scores = lax.fori_loop(
                0, 4, functools.partial(edge4, g * _L),
                jnp.zeros((_L,), jnp.float32))
            out_v[pl.ds(i * _C + g * _L, _L)] = scores * (1.0 / _D)
            return 0

        lax.fori_loop(0, _C // _L, group_body, 0)
        tail_base = (_C // _L) * _L
        scores = lax.fori_loop(
            0, 2, functools.partial(edge4, tail_base),
            jnp.zeros((_L,), jnp.float32))
        out_v[pl.ds(i * _C + tail_base, _L)] = scores * (1.0 / _D)

    last = jnp.int32(_NCHUNK - 1)
    fire_idx(0, 0)
    fire_idx(1, 1)
    wait_idx(0, 0)
    fire_g(0)
    wait_idx(1, 1)
    fire_g(1)

    def pair_body(u, _):
        for b in range(2):
            i = u * 2 + b
            nxt = jnp.minimum(i + 2, last)
            wait_g(b)
            fire_idx(nxt, b)
            compute(i, b)
            wait_idx(nxt, b)
            fire_g(b)
        return 0

    lax.fori_loop(0, _NCHUNK // 2, pair_body, 0)
    wait_g(0)
    wait_g(1)
    pltpu.sync_copy(out_v, out_hbm.at[wid])


@jax.jit
def _sc_score(node_embeddings, idx_all, rel_weight):
    mesh = plsc.VectorSubcoreMesh(core_axis_name="c", subcore_axis_name="s")
    kfn = functools.partial(
        pl.kernel,
        mesh=mesh,
        out_type=jax.ShapeDtypeStruct((_NW, _OPAD), jnp.float32),
        scratch_types=[
            pltpu.VMEM_SHARED((_N_NODES, _D), jnp.float32),
            pltpu.VMEM_SHARED((_N_REL, _D), jnp.float32),
            pltpu.VMEM((3, _C), jnp.int32),
            pltpu.VMEM((3, _C), jnp.int32),
            pltpu.VMEM((_C, _D), jnp.float32),
            pltpu.VMEM((_C, _D), jnp.float32),
            pltpu.VMEM((_C, _D), jnp.float32),
            pltpu.VMEM((_C, _D), jnp.float32),
            pltpu.VMEM((_C, _D), jnp.float32),
            pltpu.VMEM((_C, _D), jnp.float32),
            pltpu.VMEM((_OPAD,), jnp.float32),
            pltpu.SemaphoreType.DMA,
            pltpu.SemaphoreType.DMA,
            pltpu.SemaphoreType.DMA,
            pltpu.SemaphoreType.DMA,
            pltpu.SemaphoreType.DMA,
            pltpu.SemaphoreType.DMA,
            pltpu.SemaphoreType.DMA,
            pltpu.SemaphoreType.DMA,
        ],
    )(_sc_body)
    return kfn(node_embeddings, idx_all, rel_weight)


def kernel(node_embeddings, edge_index, relation_type, rel_weight):
    head = edge_index[0].reshape(_NW, _NCHUNK, _C)
    tail = edge_index[1].reshape(_NW, _NCHUNK, _C)
    rel_type = relation_type.astype(jnp.int32).reshape(_NW, _NCHUNK, _C)
    idx_all = jnp.stack([head, tail, rel_type], axis=2)
    out = _sc_score(node_embeddings, idx_all, rel_weight)
    return out[:, :_EW].reshape(_E)


# speedup vs baseline: 10.6781x; 1.2131x over previous
import functools

import jax
import jax.numpy as jnp
from jax import lax
from jax.experimental import pallas as pl
from jax.experimental.pallas import tpu as pltpu
from jax.experimental.pallas import tpu_sc as plsc

_N_NODES = 10000
_D = 128
_N_REL = 16
_E = 320000

_L = 16
_NC = 2
_NS = 16
_NW = _NC * _NS
_EW = _E // _NW
_C = 80
_NCHUNK = _EW // _C
_DP = _D // 2
_QCH = _DP // _L


def _sc_body(node_hbm, idx_hbm, relw_hbm, out_hbm,
             node_sh, relw_sh,
             ibuf0, ibuf1,
             hrows0, trows0, rrows0, hrows1, trows1, rrows1,
             out_v,
             si0, si1, sh0, st0, sr0, sh1, st1, sr1):
    wid = lax.axis_index("s") * _NC + lax.axis_index("c")
    sid = lax.axis_index("s")
    lane = lax.iota(jnp.int32, _L)
    perms = [jnp.bitwise_xor(lane, jnp.int32(1 << b)) for b in range(4)]
    gdn = lax.GatherDimensionNumbers(
        offset_dims=(), collapsed_slice_dims=(0,), start_index_map=(0,))

    def _permute(x, p):
        return lax.gather(x, p[:, None], gdn, (1,),
                          mode=lax.GatherScatterMode.PROMISE_IN_BOUNDS)

    ibufs = (ibuf0, ibuf1)
    isems = (si0, si1)
    rows = ((hrows0, trows0, rrows0), (hrows1, trows1, rrows1))
    gsems = ((sh0, st0, sr0), (sh1, st1, sr1))

    @pl.when(sid == 0)
    def _():
        pltpu.sync_copy(node_hbm, node_sh)
        pltpu.sync_copy(relw_hbm, relw_sh)

    plsc.subcore_barrier()

    def fire_idx(i, b):
        pltpu.async_copy(idx_hbm.at[wid, i], ibufs[b], isems[b])

    def wait_idx(i, b):
        pltpu.make_async_copy(idx_hbm.at[wid, i], ibufs[b], isems[b]).wait()

    def fire_g(b):
        hr, tr, rr = rows[b]
        sh, st, sr = gsems[b]
        ib = ibufs[b]
        pltpu.async_copy(node_sh.at[ib.at[0]], hr, sh)
        pltpu.async_copy(node_sh.at[ib.at[1]], tr, st)
        pltpu.async_copy(relw_sh.at[ib.at[2]], rr, sr)

    def wait_g(b):
        hr, tr, rr = rows[b]
        sh, st, sr = gsems[b]
        ib = ibufs[b]
        pltpu.make_async_copy(node_sh.at[ib.at[0]], hr, sh).wait()
        pltpu.make_async_copy(node_sh.at[ib.at[1]], tr, st).wait()
        pltpu.make_async_copy(relw_sh.at[ib.at[2]], rr, sr).wait()

    hi_mask = jnp.full((_L,), jnp.int32(-65536))

    def _two_f32(u):
        a = lax.bitcast_convert_type(jnp.left_shift(u, 16), jnp.float32)
        c = lax.bitcast_convert_type(jnp.bitwise_and(u, hi_mask), jnp.float32)
        return a, c

    def compute(i, b):
        hr, tr, rr = rows[b]

        def edge4(base_row, q, scores):
            for j in range(4):
                k = q * 4 + j
                row = base_row + k
                ps = []
                for qq in range(_QCH):
                    h0, h1 = _two_f32(hr[row, pl.ds(qq * _L, _L)])
                    r0, r1 = _two_f32(rr[row, pl.ds(qq * _L, _L)])
                    t0, t1 = _two_f32(tr[row, pl.ds(qq * _L, _L)])
                    ps.append(h0 * r0 * t0)
                    ps.append(h1 * r1 * t1)
                while len(ps) > 1:
                    ps = [ps[m] + ps[m + 1] for m in range(0, len(ps), 2)]
                acc = ps[0]
                for p in perms:
                    acc = acc + _permute(acc, p)
                scores = jnp.where(lane == k, acc, scores)
            return scores

        def group_body(g, _):
            scores = lax.fori_loop(
                0, 4, functools.partial(edge4, g * _L),
                jnp.zeros((_L,), jnp.float32))
            out_v[pl.ds(i * _C + g * _L, _L)] = scores * (1.0 / _D)
            return 0

        lax.fori_loop(0, _C // _L, group_body, 0)

    last = jnp.int32(_NCHUNK - 1)
    fire_idx(0, 0)
    fire_idx(1, 1)
    wait_idx(0, 0)
    fire_g(0)
    wait_idx(1, 1)
    fire_g(1)

    def pair_body(u, _):
        for b in range(2):
            i = u * 2 + b
            nxt = jnp.minimum(i + 2, last)
            wait_g(b)
            fire_idx(nxt, b)
            compute(i, b)
            wait_idx(nxt, b)
            fire_g(b)
        return 0

    lax.fori_loop(0, (_NCHUNK - 1) // 2, pair_body, 0)
    wait_g(0)
    compute(_NCHUNK - 1, 0)
    wait_g(1)
    pltpu.sync_copy(out_v, out_hbm.at[wid])


@jax.jit
def _sc_score(node_bf, idx_all, relw_bf):
    mesh = plsc.VectorSubcoreMesh(core_axis_name="c", subcore_axis_name="s")
    kfn = functools.partial(
        pl.kernel,
        mesh=mesh,
        out_type=jax.ShapeDtypeStruct((_NW, _EW), jnp.float32),
        scratch_types=[
            pltpu.VMEM_SHARED((_N_NODES, _DP), jnp.int32),
            pltpu.VMEM_SHARED((_N_REL, _DP), jnp.int32),
            pltpu.VMEM((3, _C), jnp.int32),
            pltpu.VMEM((3, _C), jnp.int32),
            pltpu.VMEM((_C, _DP), jnp.int32),
            pltpu.VMEM((_C, _DP), jnp.int32),
            pltpu.VMEM((_C, _DP), jnp.int32),
            pltpu.VMEM((_C, _DP), jnp.int32),
            pltpu.VMEM((_C, _DP), jnp.int32),
            pltpu.VMEM((_C, _DP), jnp.int32),
            pltpu.VMEM((_EW,), jnp.float32),
            pltpu.SemaphoreType.DMA,
            pltpu.SemaphoreType.DMA,
            pltpu.SemaphoreType.DMA,
            pltpu.SemaphoreType.DMA,
            pltpu.SemaphoreType.DMA,
            pltpu.SemaphoreType.DMA,
            pltpu.SemaphoreType.DMA,
            pltpu.SemaphoreType.DMA,
        ],
    )(_sc_body)
    return kfn(node_bf, idx_all, relw_bf)


def kernel(node_embeddings, edge_index, relation_type, rel_weight):
    head = edge_index[0].reshape(_NW, _NCHUNK, _C)
    tail = edge_index[1].reshape(_NW, _NCHUNK, _C)
    rel_type = relation_type.astype(jnp.int32).reshape(_NW, _NCHUNK, _C)
    idx_all = jnp.stack([head, tail, rel_type], axis=2)
    node_bf = node_embeddings.astype(jnp.bfloat16)
    relw_bf = rel_weight.astype(jnp.bfloat16)
    node_i32 = lax.bitcast_convert_type(
        node_bf.reshape(_N_NODES, _DP, 2), jnp.int32)
    relw_i32 = lax.bitcast_convert_type(
        relw_bf.reshape(_N_REL, _DP, 2), jnp.int32)
    out = _sc_score(node_i32, idx_all, relw_i32)
    return out.reshape(_E)
